# Initial kernel scaffold; baseline (speedup 1.0000x reference)
import jax
import jax.numpy as jnp
from jax.experimental import pallas as pl


def kernel(x, adj, W, b):
    raise NotImplementedError("write your pallas kernel here")


# speedup vs baseline: 11.3640x; 11.3640x over previous
import functools

import jax
import jax.numpy as jnp
from jax import lax
from jax.experimental import pallas as pl
from jax.experimental.pallas import tpu as pltpu
from jax.experimental.pallas import tpu_sc as plsc

N_NODES = 10000
N_EDGES = 320000
NFEAT = 128
NCLASS = 16

NC = 2
NS = 16
BLK = 128
NBLK_TOTAL = -(-N_EDGES // BLK)
NBLK_PAD = -(-NBLK_TOTAL // (NC * NS * 8)) * (NC * NS * 8)
NBLK_PT = NBLK_PAD // (NC * NS)
E_PAD = NBLK_PAD * BLK

N_PAD = 10240
ROWS_PT = N_PAD // NS
TRASH = 10200
DUMP = 64

_mesh = plsc.VectorSubcoreMesh(core_axis_name="c", subcore_axis_name="s")


@functools.partial(
    pl.kernel,
    mesh=_mesh,
    out_type=jax.ShapeDtypeStruct((NC, N_PAD), jnp.float32),
    scratch_types=[
        pltpu.VMEM((NBLK_PT, BLK), jnp.int32),
        pltpu.VMEM((BLK,), jnp.float32),
        pltpu.VMEM((ROWS_PT,), jnp.float32),
        pltpu.VMEM_SHARED((N_PAD,), jnp.float32),
    ],
)
def _sc_degree(col_hbm, out_hbm, cidx_v, ones_v, bounce_v, deg_sh):
    cid = lax.axis_index("c")
    sid = lax.axis_index("s")
    wid = cid * NS + sid

    def fill(i, _):
        ones_v[pl.ds(i * 16, 16)] = jnp.ones((16,), jnp.float32)
        return _
    lax.fori_loop(0, BLK // 16, fill, None)

    def zero(i, _):
        bounce_v[pl.ds(i * 16, 16)] = jnp.zeros((16,), jnp.float32)
        return _
    lax.fori_loop(0, ROWS_PT // 16, zero, None)

    pltpu.sync_copy(bounce_v, deg_sh.at[pl.ds(sid * ROWS_PT, ROWS_PT)])
    plsc.subcore_barrier()

    pltpu.sync_copy(col_hbm.at[pl.ds(wid * NBLK_PT, NBLK_PT)], cidx_v)

    def body(j, _):
        pltpu.sync_copy(ones_v, deg_sh.at[cidx_v.at[j]], add=True)
        return _
    lax.fori_loop(0, NBLK_PT, body, None)

    plsc.subcore_barrier()

    pltpu.sync_copy(deg_sh.at[pl.ds(sid * ROWS_PT, ROWS_PT)], bounce_v)
    pltpu.sync_copy(bounce_v, out_hbm.at[cid, pl.ds(sid * ROWS_PT, ROWS_PT)])


@functools.partial(
    pl.kernel,
    mesh=_mesh,
    out_type=jax.ShapeDtypeStruct((NC, N_PAD, NFEAT), jnp.float32),
    scratch_types=[
        pltpu.VMEM((NBLK_PT, BLK), jnp.int32),
        pltpu.VMEM((NBLK_PT, BLK), jnp.int32),
        pltpu.VMEM((BLK, NFEAT), jnp.float32),
        pltpu.VMEM((DUMP, NFEAT), jnp.float32),
        pltpu.VMEM_SHARED((N_PAD, NFEAT), jnp.float32),
        pltpu.SemaphoreType.DMA,
    ],
)
def _sc_spmm(row_hbm, col_hbm, z_hbm, out_hbm,
             ridx_v, cidx_v, msg_v, bounce_v, acc_sh, sem):
    cid = lax.axis_index("c")
    sid = lax.axis_index("s")
    wid = cid * NS + sid

    def zero(i, _):
        def zrow(j, _):
            bounce_v[i, pl.ds(j * 16, 16)] = jnp.zeros((16,), jnp.float32)
            return _
        lax.fori_loop(0, NFEAT // 16, zrow, None)
        return _
    lax.fori_loop(0, DUMP, zero, None)

    def zacc(k, _):
        pltpu.sync_copy(
            bounce_v, acc_sh.at[pl.ds(sid * ROWS_PT + k * DUMP, DUMP), :])
        return _
    lax.fori_loop(0, ROWS_PT // DUMP, zacc, None)
    plsc.subcore_barrier()

    pltpu.sync_copy(row_hbm.at[pl.ds(wid * NBLK_PT, NBLK_PT)], ridx_v)
    pltpu.sync_copy(col_hbm.at[pl.ds(wid * NBLK_PT, NBLK_PT)], cidx_v)

    def body(j, _):
        pltpu.async_copy(z_hbm.at[ridx_v.at[j]], msg_v, sem).wait()
        pltpu.sync_copy(msg_v, acc_sh.at[cidx_v.at[j]], add=True)
        return _
    lax.fori_loop(0, NBLK_PT, body, None)

    plsc.subcore_barrier()

    def dump(k, _):
        base = sid * ROWS_PT + k * DUMP
        pltpu.sync_copy(acc_sh.at[pl.ds(base, DUMP), :], bounce_v)
        pltpu.sync_copy(bounce_v, out_hbm.at[cid, pl.ds(base, DUMP), :])
        return _
    lax.fori_loop(0, ROWS_PT // DUMP, dump, None)


_RB = 1000


def _tc_prep_body(x_ref, c0_ref, c1_ref, z_ref, dis_ref):
    deg = c0_ref[...] + c1_ref[...] + 1.0
    dis = lax.rsqrt(deg)
    z_ref[...] = x_ref[...] * dis
    dis_ref[...] = dis


def _tc_prep(x, c0, c1):
    return pl.pallas_call(
        _tc_prep_body,
        grid=(N_NODES // _RB,),
        in_specs=[
            pl.BlockSpec((_RB, NFEAT), lambda i: (i, 0)),
            pl.BlockSpec((_RB, 1), lambda i: (i, 0)),
            pl.BlockSpec((_RB, 1), lambda i: (i, 0)),
        ],
        out_specs=[
            pl.BlockSpec((_RB, NFEAT), lambda i: (i, 0)),
            pl.BlockSpec((_RB, 1), lambda i: (i, 0)),
        ],
        out_shape=[
            jax.ShapeDtypeStruct((N_NODES, NFEAT), jnp.float32),
            jax.ShapeDtypeStruct((N_NODES, 1), jnp.float32),
        ],
    )(x, c0, c1)


def _tc_final_body(p0_ref, p1_ref, z_ref, dis_ref, w_ref, b_ref, o_ref):
    s = (p0_ref[...] + p1_ref[...] + z_ref[...]) * dis_ref[...]
    o_ref[...] = jnp.dot(s, w_ref[...],
                         preferred_element_type=jnp.float32) + b_ref[...]


def _tc_final(p0, p1, z, dis, W, b2d):
    return pl.pallas_call(
        _tc_final_body,
        grid=(N_NODES // _RB,),
        in_specs=[
            pl.BlockSpec((_RB, NFEAT), lambda i: (i, 0)),
            pl.BlockSpec((_RB, NFEAT), lambda i: (i, 0)),
            pl.BlockSpec((_RB, NFEAT), lambda i: (i, 0)),
            pl.BlockSpec((_RB, 1), lambda i: (i, 0)),
            pl.BlockSpec((NFEAT, NCLASS), lambda i: (0, 0)),
            pl.BlockSpec((1, NCLASS), lambda i: (0, 0)),
        ],
        out_specs=pl.BlockSpec((_RB, NCLASS), lambda i: (i, 0)),
        out_shape=jax.ShapeDtypeStruct((N_NODES, NCLASS), jnp.float32),
    )(p0, p1, z, dis, W, b2d)


def kernel(x, adj, W, b):
    row = adj[0].astype(jnp.int32)
    col = adj[1].astype(jnp.int32)

    pad = E_PAD - N_EDGES
    colp = jnp.concatenate(
        [col, jnp.full((pad,), TRASH, jnp.int32)]).reshape(NBLK_PAD, BLK)
    rowp = jnp.concatenate(
        [row, jnp.zeros((pad,), jnp.int32)]).reshape(NBLK_PAD, BLK)

    cnt = _sc_degree(colp)
    c0 = cnt[0, :N_NODES].reshape(N_NODES, 1)
    c1 = cnt[1, :N_NODES].reshape(N_NODES, 1)

    z, dis = _tc_prep(x, c0, c1)

    p = _sc_spmm(rowp, colp, z)

    return _tc_final(p[0, :N_NODES], p[1, :N_NODES], z, dis,
                     W, b.reshape(1, NCLASS))
